# Initial kernel scaffold; baseline (speedup 1.0000x reference)
import jax
import jax.numpy as jnp
from jax.experimental import pallas as pl


def kernel(node_attr, embedding_table):
    raise NotImplementedError("write your pallas kernel here")


# speedup vs baseline: 1.4151x; 1.4151x over previous
import functools

import jax
import jax.numpy as jnp
from jax import lax
from jax.experimental import pallas as pl
from jax.experimental.pallas import tpu as pltpu
from jax.experimental.pallas import tpu_sc as plsc

NTYPES = 95
DIM = 128
N_NODES = 100000

NW = 32
BPW = 3120
MAIN = NW * BPW
CH = 120
NCH = BPW // CH
TAIL = N_NODES - MAIN
TAIL_PER_W = 8
TAIL_WORKERS = TAIL // TAIL_PER_W

_mesh = plsc.VectorSubcoreMesh(core_axis_name="c", subcore_axis_name="s")


@functools.partial(
    pl.kernel,
    out_type=jax.ShapeDtypeStruct((N_NODES, DIM), jnp.float32),
    mesh=_mesh,
    scratch_types=[
        pltpu.VMEM((CH,), jnp.int32),
        pltpu.VMEM((CH, DIM), jnp.float32),
        pltpu.VMEM((TAIL_PER_W,), jnp.int32),
        pltpu.VMEM((TAIL_PER_W, DIM), jnp.float32),
        pltpu.SemaphoreType.DMA,
    ],
)
def _emb_lookup(idx_hbm, table_hbm, out_hbm, idx_v, rows_v, tidx_v, trows_v, sem):
    wid = lax.axis_index("s") * 2 + lax.axis_index("c")
    base = pl.multiple_of(wid * BPW, 8)

    def body(c, carry):
        off = pl.multiple_of(base + c * CH, 8)
        pltpu.sync_copy(idx_hbm.at[pl.ds(off, CH)], idx_v)
        pltpu.async_copy(table_hbm.at[idx_v], rows_v, sem).wait()
        pltpu.sync_copy(rows_v, out_hbm.at[pl.ds(off, CH)])
        return carry

    lax.fori_loop(0, NCH, body, 0)

    @pl.when(wid < TAIL_WORKERS)
    def _tail():
        toff = pl.multiple_of(MAIN + wid * TAIL_PER_W, 8)
        pltpu.sync_copy(idx_hbm.at[pl.ds(toff, TAIL_PER_W)], tidx_v)
        pltpu.async_copy(table_hbm.at[tidx_v], trows_v, sem).wait()
        pltpu.sync_copy(trows_v, out_hbm.at[pl.ds(toff, TAIL_PER_W)])


def kernel(node_attr, embedding_table):
    idx = node_attr.astype(jnp.int32)
    return _emb_lookup(idx, embedding_table)


# speedup vs baseline: 1.4345x; 1.0137x over previous
import functools

import jax
import jax.numpy as jnp
from jax import lax
from jax.experimental import pallas as pl
from jax.experimental.pallas import tpu as pltpu
from jax.experimental.pallas import tpu_sc as plsc

NTYPES = 95
DIM = 128
N_NODES = 100000

NW = 32
BPW = 3120
MAIN = NW * BPW
CH = 120
NCH = BPW // CH
TAIL = N_NODES - MAIN
TAIL_PER_W = 8
TAIL_WORKERS = TAIL // TAIL_PER_W

_mesh = plsc.VectorSubcoreMesh(core_axis_name="c", subcore_axis_name="s")


@functools.partial(
    pl.kernel,
    out_type=jax.ShapeDtypeStruct((N_NODES, DIM), jnp.float32),
    mesh=_mesh,
    scratch_types=[
        pltpu.VMEM((NCH, CH), jnp.int32),
        pltpu.VMEM((CH, DIM), jnp.float32),
        pltpu.VMEM((CH, DIM), jnp.float32),
        pltpu.VMEM((TAIL_PER_W,), jnp.int32),
        pltpu.VMEM((TAIL_PER_W, DIM), jnp.float32),
        pltpu.SemaphoreType.DMA,
        pltpu.SemaphoreType.DMA,
        pltpu.SemaphoreType.DMA,
        pltpu.SemaphoreType.DMA,
    ],
)
def _emb_lookup(idx3d_hbm, tail_hbm, table_hbm, out_hbm,
                idx_v, rows0, rows1, tidx_v, trows_v,
                gsem0, gsem1, ssem0, ssem1):
    wid = lax.axis_index("s") * 2 + lax.axis_index("c")
    base = pl.multiple_of(wid * BPW, 8)

    pltpu.sync_copy(idx3d_hbm.at[wid], idx_v)

    def gather(c, rows, gsem):
        pltpu.async_copy(table_hbm.at[idx_v.at[c]], rows, gsem)

    def store(c, rows, ssem):
        off = pl.multiple_of(base + c * CH, 8)
        pltpu.async_copy(rows, out_hbm.at[pl.ds(off, CH)], ssem)

    def wait_gather(rows, gsem):
        pltpu.make_async_copy(table_hbm.at[idx_v.at[0]], rows, gsem).wait()

    def wait_store(rows, ssem):
        pltpu.make_async_copy(rows, out_hbm.at[pl.ds(0, CH)], ssem).wait()

    gather(0, rows0, gsem0)

    def body(g, carry):
        c0 = 2 * g

        @pl.when(g > 0)
        def _():
            wait_store(rows1, ssem1)
        gather(c0 + 1, rows1, gsem1)

        wait_gather(rows0, gsem0)
        store(c0, rows0, ssem0)

        @pl.when(g < NCH // 2 - 1)
        def _():
            wait_store(rows0, ssem0)
            gather(c0 + 2, rows0, gsem0)

        wait_gather(rows1, gsem1)
        store(c0 + 1, rows1, ssem1)
        return carry

    lax.fori_loop(0, NCH // 2, body, 0)
    wait_store(rows0, ssem0)
    wait_store(rows1, ssem1)

    @pl.when(wid < TAIL_WORKERS)
    def _tail():
        toff = pl.multiple_of(MAIN + wid * TAIL_PER_W, 8)
        pltpu.sync_copy(tail_hbm.at[pl.ds(wid * TAIL_PER_W, TAIL_PER_W)], tidx_v)
        pltpu.async_copy(table_hbm.at[tidx_v], trows_v, gsem0).wait()
        pltpu.sync_copy(trows_v, out_hbm.at[pl.ds(toff, TAIL_PER_W)])


def kernel(node_attr, embedding_table):
    idx = node_attr.astype(jnp.int32)
    idx_main = idx[:MAIN].reshape(NW, NCH, CH)
    idx_tail = idx[MAIN:]
    return _emb_lookup(idx_main, idx_tail, embedding_table)


# speedup vs baseline: 5.3906x; 3.7578x over previous
import functools

import jax
import jax.numpy as jnp
from jax import lax
from jax.experimental import pallas as pl
from jax.experimental.pallas import tpu as pltpu
from jax.experimental.pallas import tpu_sc as plsc

NTYPES = 95
DIM = 128
N_NODES = 100000

NW = 32
BPW = 3120
MAIN = NW * BPW
CH = 120
NCH = BPW // CH
TAIL = N_NODES - MAIN
TAIL_PER_W = 8
TAIL_WORKERS = TAIL // TAIL_PER_W

_mesh = plsc.VectorSubcoreMesh(core_axis_name="c", subcore_axis_name="s")


@functools.partial(
    pl.kernel,
    out_type=jax.ShapeDtypeStruct((N_NODES, DIM), jnp.float32),
    mesh=_mesh,
    scratch_types=[
        pltpu.VMEM((NCH, CH), jnp.int32),
        pltpu.VMEM((CH, DIM), jnp.float32),
        pltpu.VMEM((CH, DIM), jnp.float32),
        pltpu.VMEM((TAIL_PER_W,), jnp.int32),
        pltpu.VMEM((TAIL_PER_W, DIM), jnp.float32),
        pltpu.VMEM((NTYPES, DIM), jnp.float32),
        pltpu.VMEM_SHARED((NTYPES, DIM), jnp.float32),
        pltpu.SemaphoreType.DMA,
        pltpu.SemaphoreType.DMA,
        pltpu.SemaphoreType.DMA,
        pltpu.SemaphoreType.DMA,
    ],
)
def _emb_lookup(idx3d_hbm, tail_hbm, table_hbm, out_hbm,
                idx_v, rows0, rows1, tidx_v, trows_v, table_l, table_sh,
                gsem0, gsem1, ssem0, ssem1):
    wid = lax.axis_index("s") * 2 + lax.axis_index("c")
    base = pl.multiple_of(wid * BPW, 8)

    @pl.when(lax.axis_index("s") == 0)
    def _stage():
        pltpu.sync_copy(table_hbm, table_l)
        pltpu.sync_copy(table_l, table_sh)
    plsc.subcore_barrier()

    pltpu.sync_copy(idx3d_hbm.at[wid], idx_v)

    def gather(c, rows, gsem):
        pltpu.async_copy(table_sh.at[idx_v.at[c]], rows, gsem)

    def store(c, rows, ssem):
        off = pl.multiple_of(base + c * CH, 8)
        pltpu.async_copy(rows, out_hbm.at[pl.ds(off, CH)], ssem)

    def wait_gather(rows, gsem):
        pltpu.make_async_copy(table_sh.at[idx_v.at[0]], rows, gsem).wait()

    def wait_store(rows, ssem):
        pltpu.make_async_copy(rows, out_hbm.at[pl.ds(0, CH)], ssem).wait()

    gather(0, rows0, gsem0)

    def body(g, carry):
        c0 = 2 * g

        @pl.when(g > 0)
        def _():
            wait_store(rows1, ssem1)
        gather(c0 + 1, rows1, gsem1)

        wait_gather(rows0, gsem0)
        store(c0, rows0, ssem0)

        @pl.when(g < NCH // 2 - 1)
        def _():
            wait_store(rows0, ssem0)
            gather(c0 + 2, rows0, gsem0)

        wait_gather(rows1, gsem1)
        store(c0 + 1, rows1, ssem1)
        return carry

    lax.fori_loop(0, NCH // 2, body, 0)
    wait_store(rows0, ssem0)
    wait_store(rows1, ssem1)

    @pl.when(wid < TAIL_WORKERS)
    def _tail():
        toff = pl.multiple_of(MAIN + wid * TAIL_PER_W, 8)
        pltpu.sync_copy(tail_hbm.at[pl.ds(wid * TAIL_PER_W, TAIL_PER_W)], tidx_v)
        pltpu.async_copy(table_hbm.at[tidx_v], trows_v, gsem0).wait()
        pltpu.sync_copy(trows_v, out_hbm.at[pl.ds(toff, TAIL_PER_W)])


def kernel(node_attr, embedding_table):
    idx = node_attr.astype(jnp.int32)
    idx_main = idx[:MAIN].reshape(NW, NCH, CH)
    idx_tail = idx[MAIN:]
    return _emb_lookup(idx_main, idx_tail, embedding_table)


# speedup vs baseline: 5.5566x; 1.0308x over previous
import functools

import jax
import jax.numpy as jnp
from jax import lax
from jax.experimental import pallas as pl
from jax.experimental.pallas import tpu as pltpu
from jax.experimental.pallas import tpu_sc as plsc

NTYPES = 95
DIM = 128
N_NODES = 100000

NW = 32
BPW = 3120
MAIN = NW * BPW
CH = 104
NCH = BPW // CH
NBUF = 5
LEAD = 2
NGRP = NCH // NBUF
TAIL = N_NODES - MAIN
TAIL_PER_W = 8
TAIL_WORKERS = TAIL // TAIL_PER_W

_mesh = plsc.VectorSubcoreMesh(core_axis_name="c", subcore_axis_name="s")


@functools.partial(
    pl.kernel,
    out_type=jax.ShapeDtypeStruct((N_NODES, DIM), jnp.float32),
    mesh=_mesh,
    scratch_types=[
        pltpu.VMEM((NCH, CH), jnp.int32),
        [pltpu.VMEM((CH, DIM), jnp.float32) for _ in range(NBUF)],
        pltpu.VMEM((TAIL_PER_W,), jnp.int32),
        pltpu.VMEM((TAIL_PER_W, DIM), jnp.float32),
        pltpu.VMEM((NTYPES, DIM), jnp.float32),
        pltpu.VMEM_SHARED((NTYPES, DIM), jnp.float32),
        [pltpu.SemaphoreType.DMA for _ in range(NBUF)],
        [pltpu.SemaphoreType.DMA for _ in range(NBUF)],
    ],
)
def _emb_lookup(idx3d_hbm, tail_hbm, table_hbm, out_hbm,
                idx_v, rows, tidx_v, trows_v, table_l, table_sh,
                gsem, ssem):
    wid = lax.axis_index("s") * 2 + lax.axis_index("c")
    base = pl.multiple_of(wid * BPW, 8)

    @pl.when(lax.axis_index("s") == 0)
    def _stage():
        pltpu.sync_copy(table_hbm, table_l)
        pltpu.sync_copy(table_l, table_sh)
    plsc.subcore_barrier()

    pltpu.sync_copy(idx3d_hbm.at[wid], idx_v)

    def gather(c, b):
        pltpu.async_copy(table_sh.at[idx_v.at[c]], rows[b], gsem[b])

    def store(c, b):
        off = pl.multiple_of(base + c * CH, 8)
        pltpu.async_copy(rows[b], out_hbm.at[pl.ds(off, CH)], ssem[b])

    def wait_gather(b):
        pltpu.make_async_copy(table_sh.at[idx_v.at[0]], rows[b], gsem[b]).wait()

    def wait_store(b):
        pltpu.make_async_copy(rows[b], out_hbm.at[pl.ds(0, CH)], ssem[b]).wait()

    for b in range(LEAD):
        gather(b, b)

    def body(g, carry):
        for b in range(NBUF):
            c = g * NBUF + b
            bg = (b + LEAD) % NBUF
            if b + LEAD - NBUF >= 0:
                wait_store(bg)
            else:
                @pl.when(g > 0)
                def _():
                    wait_store(bg)

            @pl.when(c + LEAD < NCH)
            def _():
                gather(c + LEAD, bg)

            wait_gather(b)
            store(c, b)
        return carry

    lax.fori_loop(0, NGRP, body, 0)
    for k in range(NCH - (NBUF - LEAD), NCH):
        wait_store(k % NBUF)

    @pl.when(wid < TAIL_WORKERS)
    def _tail():
        toff = pl.multiple_of(MAIN + wid * TAIL_PER_W, 8)
        pltpu.sync_copy(tail_hbm.at[pl.ds(wid * TAIL_PER_W, TAIL_PER_W)], tidx_v)
        pltpu.async_copy(table_sh.at[tidx_v], trows_v, gsem[0]).wait()
        pltpu.sync_copy(trows_v, out_hbm.at[pl.ds(toff, TAIL_PER_W)])


def kernel(node_attr, embedding_table):
    idx = node_attr.astype(jnp.int32)
    idx_main = idx[:MAIN].reshape(NW, NCH, CH)
    idx_tail = idx[MAIN:]
    return _emb_lookup(idx_main, idx_tail, embedding_table)
